# Initial kernel scaffold; baseline (speedup 1.0000x reference)
import functools

import jax
import jax.numpy as jnp
from jax import lax
from jax.experimental import pallas as pl
from jax.experimental.pallas import tpu as pltpu
from jax.experimental.pallas import tpu_sc as plsc

N = 10000
E = 320000
D = 128

NC = 2
NS = 16
NW = NC * NS
EPT = E // NW
CH = 80
NCH = EPT // CH
RPT = N // NS

_mesh = plsc.VectorSubcoreMesh(core_axis_name="c", subcore_axis_name="s")


@functools.partial(
    pl.kernel,
    out_type=jax.ShapeDtypeStruct((2, NC, N, 16), jnp.float32),
    mesh=_mesh,
    scratch_types=[
        pltpu.VMEM((CH, 16), jnp.float32),
        pltpu.VMEM((CH,), jnp.int32),
        pltpu.VMEM((CH,), jnp.int32),
        pltpu.VMEM_SHARED((N, 16), jnp.float32),
        pltpu.VMEM_SHARED((N, 16), jnp.float32),
    ],
)
def _deg_kernel(src_hbm, dst_hbm, onehot_hbm, zeros_hbm, out_hbm,
                oh_v, sidx, didx, dout_sh, din_sh):
    c = lax.axis_index("c")
    s = lax.axis_index("s")
    base = (s * NC + c) * EPT
    pltpu.sync_copy(onehot_hbm, oh_v)
    rows = pl.ds(s * RPT, RPT)
    pltpu.sync_copy(zeros_hbm, dout_sh.at[rows])
    pltpu.sync_copy(zeros_hbm, din_sh.at[rows])
    plsc.subcore_barrier()

    def body(i, carry):
        off = base + i * CH
        pltpu.sync_copy(src_hbm.at[pl.ds(off, CH)], sidx)
        pltpu.sync_copy(dst_hbm.at[pl.ds(off, CH)], didx)
        pltpu.sync_copy(oh_v, dout_sh.at[sidx], add=True)
        pltpu.sync_copy(oh_v, din_sh.at[didx], add=True)
        return carry

    lax.fori_loop(0, NCH, body, 0)
    plsc.subcore_barrier()
    pltpu.sync_copy(dout_sh.at[rows], out_hbm.at[0, c, rows])
    pltpu.sync_copy(din_sh.at[rows], out_hbm.at[1, c, rows])


@functools.partial(
    pl.kernel,
    out_type=jax.ShapeDtypeStruct((NC, N, D), jnp.float32),
    mesh=_mesh,
    scratch_types=[
        pltpu.VMEM((CH,), jnp.int32),
        pltpu.VMEM((CH,), jnp.int32),
        pltpu.VMEM((CH, D), jnp.float32),
        pltpu.VMEM_SHARED((N, D), jnp.float32),
        pltpu.SemaphoreType.DMA,
    ],
)
def _scatter_kernel(y_hbm, src_hbm, dst_hbm, zeros_hbm, out_hbm,
                    sidx, didx, rows_v, agg_sh, sem):
    c = lax.axis_index("c")
    s = lax.axis_index("s")
    base = (s * NC + c) * EPT
    rows = pl.ds(s * RPT, RPT)
    pltpu.sync_copy(zeros_hbm, agg_sh.at[rows])
    plsc.subcore_barrier()

    def body(i, carry):
        off = base + i * CH
        pltpu.sync_copy(src_hbm.at[pl.ds(off, CH)], sidx)
        cp = pltpu.async_copy(y_hbm.at[sidx], rows_v, sem)
        pltpu.sync_copy(dst_hbm.at[pl.ds(off, CH)], didx)
        cp.wait()
        pltpu.sync_copy(rows_v, agg_sh.at[didx], add=True)
        return carry

    lax.fori_loop(0, NCH, body, 0)
    plsc.subcore_barrier()
    pltpu.sync_copy(agg_sh.at[rows], out_hbm.at[c, rows])


def _tc1_body(parts_ref, x_ref, w_ref, y_ref, nrm_ref):
    p = parts_ref[...]
    t_out = p[0, 0] + p[0, 1]
    t_in = p[1, 0] + p[1, 1]
    deg_out = jnp.maximum(jnp.sum(t_out, axis=1, keepdims=True), 1.0)
    deg_in = jnp.maximum(jnp.sum(t_in, axis=1, keepdims=True), 1.0)
    ns = lax.rsqrt(deg_out)
    nd = lax.rsqrt(deg_in)
    nrm_ref[:, 0:1] = ns
    nrm_ref[:, 1:2] = nd
    hw = jnp.dot(x_ref[...], w_ref[...], preferred_element_type=jnp.float32)
    y_ref[...] = hw * ns


def _tc2_body(agg_ref, nrm_ref, b_ref, w_ref, y_ref):
    agg = agg_ref[0] + agg_ref[1]
    ns = nrm_ref[:, 0:1]
    nd = nrm_ref[:, 1:2]
    h = jnp.maximum(agg * nd + b_ref[...][None, :], 0.0)
    hw = jnp.dot(h, w_ref[...], preferred_element_type=jnp.float32)
    y_ref[...] = hw * ns


def _tc3_body(agg_ref, nrm_ref, b_ref, out_ref):
    agg = agg_ref[0] + agg_ref[1]
    nd = nrm_ref[:, 1:2]
    out_ref[...] = agg * nd + b_ref[...][None, :]


_tc1 = pl.pallas_call(
    _tc1_body,
    out_shape=(
        jax.ShapeDtypeStruct((N, D), jnp.float32),
        jax.ShapeDtypeStruct((N, 2), jnp.float32),
    ),
)

_tc2 = pl.pallas_call(
    _tc2_body,
    out_shape=jax.ShapeDtypeStruct((N, D), jnp.float32),
)

_tc3 = pl.pallas_call(
    _tc3_body,
    out_shape=jax.ShapeDtypeStruct((N, D), jnp.float32),
)


@jax.jit
def kernel(in_feat, edge_index, W1, b1, W2, b2):
    src = edge_index[0]
    dst = edge_index[1]
    onehot = jnp.tile(jnp.eye(16, dtype=jnp.float32), (CH // 16, 1))
    zeros16 = jnp.zeros((RPT, 16), jnp.float32)
    zerosd = jnp.zeros((RPT, D), jnp.float32)

    parts = _deg_kernel(src, dst, onehot, zeros16)
    y1, nrm = _tc1(parts, in_feat, W1)
    agg1 = _scatter_kernel(y1, src, dst, zerosd)
    y2 = _tc2(agg1, nrm, b1, W2)
    agg2 = _scatter_kernel(y2, src, dst, zerosd)
    return _tc3(agg2, nrm, b2)


# speedup vs baseline: 7.6609x; 7.6609x over previous
import functools

import jax
import jax.numpy as jnp
from jax import lax
from jax.experimental import pallas as pl
from jax.experimental.pallas import tpu as pltpu
from jax.experimental.pallas import tpu_sc as plsc

N = 10000
NP = 10240
E = 320000
D = 128

NC = 2
NS = 16
NW = NC * NS
EPT = E // NW
CH = 80
NCH = EPT // CH
RPT = NP // NS

_mesh = plsc.VectorSubcoreMesh(core_axis_name="c", subcore_axis_name="s")


@functools.partial(
    pl.kernel,
    out_type=jax.ShapeDtypeStruct((NC, NP, D), jnp.float32),
    mesh=_mesh,
    scratch_types=[
        pltpu.VMEM((CH, D), jnp.float32),
        pltpu.VMEM((CH, D), jnp.float32),
        pltpu.VMEM((CH,), jnp.int32),
        pltpu.VMEM((CH,), jnp.int32),
        pltpu.VMEM_SHARED((NP, D), jnp.float32),
    ],
)
def _deg_kernel(src_hbm, dst_hbm, onehot_hbm, zeros_hbm, out_hbm,
                ohs_v, ohd_v, sidx, didx, deg_sh):
    c = lax.axis_index("c")
    s = lax.axis_index("s")
    base = (s * NC + c) * EPT
    pltpu.sync_copy(onehot_hbm.at[0], ohs_v)
    pltpu.sync_copy(onehot_hbm.at[1], ohd_v)
    rows = pl.ds(s * RPT, RPT)
    pltpu.sync_copy(zeros_hbm, deg_sh.at[rows])
    plsc.subcore_barrier()

    def body(i, carry):
        off = base + i * CH
        pltpu.sync_copy(src_hbm.at[pl.ds(off, CH)], sidx)
        pltpu.sync_copy(dst_hbm.at[pl.ds(off, CH)], didx)
        pltpu.sync_copy(ohs_v, deg_sh.at[sidx], add=True)
        pltpu.sync_copy(ohd_v, deg_sh.at[didx], add=True)
        return carry

    lax.fori_loop(0, NCH, body, 0)
    plsc.subcore_barrier()
    pltpu.sync_copy(deg_sh.at[rows], out_hbm.at[c, rows])


@functools.partial(
    pl.kernel,
    out_type=jax.ShapeDtypeStruct((NC, NP, D), jnp.float32),
    mesh=_mesh,
    scratch_types=[
        pltpu.VMEM((CH,), jnp.int32),
        pltpu.VMEM((CH,), jnp.int32),
        pltpu.VMEM((CH, D), jnp.float32),
        pltpu.VMEM_SHARED((NP, D), jnp.float32),
        pltpu.SemaphoreType.DMA,
    ],
)
def _scatter_kernel(y_hbm, src_hbm, dst_hbm, zeros_hbm, out_hbm,
                    sidx, didx, rows_v, agg_sh, sem):
    c = lax.axis_index("c")
    s = lax.axis_index("s")
    base = (s * NC + c) * EPT
    rows = pl.ds(s * RPT, RPT)
    pltpu.sync_copy(zeros_hbm, agg_sh.at[rows])
    plsc.subcore_barrier()

    def body(i, carry):
        off = base + i * CH
        pltpu.sync_copy(src_hbm.at[pl.ds(off, CH)], sidx)
        cp = pltpu.async_copy(y_hbm.at[sidx], rows_v, sem)
        pltpu.sync_copy(dst_hbm.at[pl.ds(off, CH)], didx)
        cp.wait()
        pltpu.sync_copy(rows_v, agg_sh.at[didx], add=True)
        return carry

    lax.fori_loop(0, NCH, body, 0)
    plsc.subcore_barrier()
    pltpu.sync_copy(agg_sh.at[rows], out_hbm.at[c, rows])


def _tc1_body(parts_ref, x_ref, w_ref, y_ref, nrm_ref):
    p = parts_ref[0] + parts_ref[1]
    t_out = p[:, 0:16]
    t_in = p[:, 16:32]
    deg_out = jnp.maximum(jnp.sum(t_out, axis=1, keepdims=True), 1.0)
    deg_in = jnp.maximum(jnp.sum(t_in, axis=1, keepdims=True), 1.0)
    ns = lax.rsqrt(deg_out)
    nd = lax.rsqrt(deg_in)
    nrm_ref[:, 0:1] = ns
    nrm_ref[:, 1:2] = nd
    hw = jnp.dot(x_ref[...], w_ref[...], preferred_element_type=jnp.float32)
    y_ref[...] = hw * ns


def _tc2_body(agg_ref, nrm_ref, b_ref, w_ref, y_ref):
    agg = agg_ref[0] + agg_ref[1]
    ns = nrm_ref[:, 0:1]
    nd = nrm_ref[:, 1:2]
    h = jnp.maximum(agg * nd + b_ref[...][None, :], 0.0)
    hw = jnp.dot(h, w_ref[...], preferred_element_type=jnp.float32)
    y_ref[...] = hw * ns


def _tc3_body(agg_ref, nrm_ref, b_ref, out_ref):
    agg = agg_ref[0] + agg_ref[1]
    nd = nrm_ref[:, 1:2]
    out_ref[...] = agg * nd + b_ref[...][None, :]


_tc1 = pl.pallas_call(
    _tc1_body,
    out_shape=(
        jax.ShapeDtypeStruct((NP, D), jnp.float32),
        jax.ShapeDtypeStruct((NP, 2), jnp.float32),
    ),
)

_tc2 = pl.pallas_call(
    _tc2_body,
    out_shape=jax.ShapeDtypeStruct((NP, D), jnp.float32),
)

_tc3 = pl.pallas_call(
    _tc3_body,
    out_shape=jax.ShapeDtypeStruct((NP, D), jnp.float32),
)


@jax.jit
def kernel(in_feat, edge_index, W1, b1, W2, b2):
    src = edge_index[0]
    dst = edge_index[1]
    x = jnp.pad(in_feat, ((0, NP - N), (0, 0)))
    eye = jnp.tile(jnp.eye(16, dtype=jnp.float32), (CH // 16, 1))
    oh_src = jnp.pad(eye, ((0, 0), (0, D - 16)))
    oh_dst = jnp.pad(eye, ((0, 0), (16, D - 32)))
    onehot = jnp.stack([oh_src, oh_dst])
    zerosd = jnp.zeros((RPT, D), jnp.float32)

    parts = _deg_kernel(src, dst, onehot, zerosd)
    y1, nrm = _tc1(parts, x, W1)
    agg1 = _scatter_kernel(y1, src, dst, zerosd)
    y2 = _tc2(agg1, nrm, b1, W2)
    agg2 = _scatter_kernel(y2, src, dst, zerosd)
    return _tc3(agg2, nrm, b2)[:N]


# speedup vs baseline: 11.8026x; 1.5406x over previous
import functools

import jax
import jax.numpy as jnp
from jax import lax
from jax.experimental import pallas as pl
from jax.experimental.pallas import tpu as pltpu
from jax.experimental.pallas import tpu_sc as plsc

N = 10000
NP = 10240
E = 320000
D = 128

NC = 2
NS = 16
NW = NC * NS
EPT = E // NW
CH = 80
NCH = EPT // CH
KD = 5
NGD = NCH // KD
KS = 4
NGS = NCH // KS
RPT = NP // NS

_mesh = plsc.VectorSubcoreMesh(core_axis_name="c", subcore_axis_name="s")


@functools.partial(
    pl.kernel,
    out_type=jax.ShapeDtypeStruct((NC, NP, D), jnp.float32),
    mesh=_mesh,
    scratch_types=(
        [pltpu.VMEM((CH, D), jnp.float32)] * 2
        + [pltpu.VMEM((CH,), jnp.int32)] * KD
        + [pltpu.VMEM((CH,), jnp.int32)] * KD
        + [
            pltpu.VMEM_SHARED((NP, D), jnp.float32),
            pltpu.SemaphoreType.DMA,
            pltpu.SemaphoreType.DMA,
        ]
    ),
)
def _deg_kernel(src_hbm, dst_hbm, onehot_hbm, zeros_hbm, out_hbm,
                ohs_v, ohd_v, *rest):
    sidx = rest[0:KD]
    didx = rest[KD:2 * KD]
    deg_sh, semi, sems = rest[2 * KD:]
    c = lax.axis_index("c")
    s = lax.axis_index("s")
    base = (s * NC + c) * EPT
    pltpu.sync_copy(onehot_hbm.at[0], ohs_v)
    pltpu.sync_copy(onehot_hbm.at[1], ohd_v)
    rows = pl.ds(s * RPT, RPT)
    pltpu.sync_copy(zeros_hbm, deg_sh.at[rows])
    plsc.subcore_barrier()

    def body(g, carry):
        off = base + g * (KD * CH)
        loads = []
        for k in range(KD):
            sl = pl.ds(off + k * CH, CH)
            loads.append(pltpu.async_copy(src_hbm.at[sl], sidx[k], semi))
            loads.append(pltpu.async_copy(dst_hbm.at[sl], didx[k], semi))
        for d in loads:
            d.wait()
        adds = []
        for k in range(KD):
            adds.append(
                pltpu.async_copy(ohs_v, deg_sh.at[sidx[k]], sems, add=True))
            adds.append(
                pltpu.async_copy(ohd_v, deg_sh.at[didx[k]], sems, add=True))
        for d in adds:
            d.wait()
        return carry

    lax.fori_loop(0, NGD, body, 0)
    plsc.subcore_barrier()
    pltpu.sync_copy(deg_sh.at[rows], out_hbm.at[c, rows])


@functools.partial(
    pl.kernel,
    out_type=jax.ShapeDtypeStruct((NC, NP, D), jnp.float32),
    mesh=_mesh,
    scratch_types=(
        [pltpu.VMEM((CH,), jnp.int32)] * KS
        + [pltpu.VMEM((CH,), jnp.int32)] * KS
        + [pltpu.VMEM((CH, D), jnp.float32)] * KS
        + [
            pltpu.VMEM_SHARED((NP, D), jnp.float32),
            pltpu.SemaphoreType.DMA,
            pltpu.SemaphoreType.DMA,
            pltpu.SemaphoreType.DMA,
        ]
    ),
)
def _scatter_kernel(y_hbm, src_hbm, dst_hbm, zeros_hbm, out_hbm, *rest):
    sidx = rest[0:KS]
    didx = rest[KS:2 * KS]
    rows_v = rest[2 * KS:3 * KS]
    agg_sh, semi, semg, sems = rest[3 * KS:]
    c = lax.axis_index("c")
    s = lax.axis_index("s")
    base = (s * NC + c) * EPT
    rows = pl.ds(s * RPT, RPT)
    pltpu.sync_copy(zeros_hbm, agg_sh.at[rows])
    plsc.subcore_barrier()

    def body(g, carry):
        off = base + g * (KS * CH)
        loads = []
        for k in range(KS):
            sl = pl.ds(off + k * CH, CH)
            loads.append(pltpu.async_copy(src_hbm.at[sl], sidx[k], semi))
            loads.append(pltpu.async_copy(dst_hbm.at[sl], didx[k], semi))
        for d in loads:
            d.wait()
        gathers = [
            pltpu.async_copy(y_hbm.at[sidx[k]], rows_v[k], semg)
            for k in range(KS)
        ]
        for d in gathers:
            d.wait()
        adds = [
            pltpu.async_copy(rows_v[k], agg_sh.at[didx[k]], sems, add=True)
            for k in range(KS)
        ]
        for d in adds:
            d.wait()
        return carry

    lax.fori_loop(0, NGS, body, 0)
    toff = base + NGS * (KS * CH)
    pltpu.sync_copy(src_hbm.at[pl.ds(toff, CH)], sidx[0])
    pltpu.async_copy(y_hbm.at[sidx[0]], rows_v[0], semg).wait()
    pltpu.sync_copy(dst_hbm.at[pl.ds(toff, CH)], didx[0])
    pltpu.sync_copy(rows_v[0], agg_sh.at[didx[0]], add=True)
    plsc.subcore_barrier()
    pltpu.sync_copy(agg_sh.at[rows], out_hbm.at[c, rows])


def _tc1_body(parts_ref, x_ref, w_ref, y_ref, nrm_ref):
    p = parts_ref[0] + parts_ref[1]
    t_out = p[:, 0:16]
    t_in = p[:, 16:32]
    deg_out = jnp.maximum(jnp.sum(t_out, axis=1, keepdims=True), 1.0)
    deg_in = jnp.maximum(jnp.sum(t_in, axis=1, keepdims=True), 1.0)
    ns = lax.rsqrt(deg_out)
    nd = lax.rsqrt(deg_in)
    nrm_ref[:, 0:1] = ns
    nrm_ref[:, 1:2] = nd
    hw = jnp.dot(x_ref[...], w_ref[...], preferred_element_type=jnp.float32)
    y_ref[...] = hw * ns


def _tc2_body(agg_ref, nrm_ref, b_ref, w_ref, y_ref):
    agg = agg_ref[0] + agg_ref[1]
    ns = nrm_ref[:, 0:1]
    nd = nrm_ref[:, 1:2]
    h = jnp.maximum(agg * nd + b_ref[...][None, :], 0.0)
    hw = jnp.dot(h, w_ref[...], preferred_element_type=jnp.float32)
    y_ref[...] = hw * ns


def _tc3_body(agg_ref, nrm_ref, b_ref, out_ref):
    agg = agg_ref[0] + agg_ref[1]
    nd = nrm_ref[:, 1:2]
    out_ref[...] = agg * nd + b_ref[...][None, :]


_tc1 = pl.pallas_call(
    _tc1_body,
    out_shape=(
        jax.ShapeDtypeStruct((NP, D), jnp.float32),
        jax.ShapeDtypeStruct((NP, 2), jnp.float32),
    ),
)

_tc2 = pl.pallas_call(
    _tc2_body,
    out_shape=jax.ShapeDtypeStruct((NP, D), jnp.float32),
)

_tc3 = pl.pallas_call(
    _tc3_body,
    out_shape=jax.ShapeDtypeStruct((NP, D), jnp.float32),
)


@jax.jit
def kernel(in_feat, edge_index, W1, b1, W2, b2):
    src = edge_index[0]
    dst = edge_index[1]
    x = jnp.pad(in_feat, ((0, NP - N), (0, 0)))
    eye = jnp.tile(jnp.eye(16, dtype=jnp.float32), (CH // 16, 1))
    oh_src = jnp.pad(eye, ((0, 0), (0, D - 16)))
    oh_dst = jnp.pad(eye, ((0, 0), (16, D - 32)))
    onehot = jnp.stack([oh_src, oh_dst])
    zerosd = jnp.zeros((RPT, D), jnp.float32)

    parts = _deg_kernel(src, dst, onehot, zerosd)
    y1, nrm = _tc1(parts, x, W1)
    agg1 = _scatter_kernel(y1, src, dst, zerosd)
    y2 = _tc2(agg1, nrm, b1, W2)
    agg2 = _scatter_kernel(y2, src, dst, zerosd)
    return _tc3(agg2, nrm, b2)[:N]


# speedup vs baseline: 12.5003x; 1.0591x over previous
import functools

import jax
import jax.numpy as jnp
from jax import lax
from jax.experimental import pallas as pl
from jax.experimental.pallas import tpu as pltpu
from jax.experimental.pallas import tpu_sc as plsc

N = 10000
NP = 10240
E = 320000
D = 128

NC = 2
NS = 16
NW = NC * NS
EPT = E // NW
CH = 80
NCH = EPT // CH
KD = 5
NGD = NCH // KD
KS = 2
NGS = NCH // (2 * KS)
RPT = NP // NS

_mesh = plsc.VectorSubcoreMesh(core_axis_name="c", subcore_axis_name="s")


@functools.partial(
    pl.kernel,
    out_type=jax.ShapeDtypeStruct((NC, NP, D), jnp.float32),
    mesh=_mesh,
    scratch_types=(
        [pltpu.VMEM((CH, D), jnp.float32)] * 2
        + [pltpu.VMEM((CH,), jnp.int32)] * KD
        + [pltpu.VMEM((CH,), jnp.int32)] * KD
        + [
            pltpu.VMEM_SHARED((NP, D), jnp.float32),
            pltpu.SemaphoreType.DMA,
            pltpu.SemaphoreType.DMA,
        ]
    ),
)
def _deg_kernel(src_hbm, dst_hbm, onehot_hbm, zeros_hbm, out_hbm,
                ohs_v, ohd_v, *rest):
    sidx = rest[0:KD]
    didx = rest[KD:2 * KD]
    deg_sh, semi, sems = rest[2 * KD:]
    c = lax.axis_index("c")
    s = lax.axis_index("s")
    base = (s * NC + c) * EPT
    pltpu.sync_copy(onehot_hbm.at[0], ohs_v)
    pltpu.sync_copy(onehot_hbm.at[1], ohd_v)
    rows = pl.ds(s * RPT, RPT)
    pltpu.sync_copy(zeros_hbm, deg_sh.at[rows])
    plsc.subcore_barrier()

    def body(g, carry):
        off = base + g * (KD * CH)
        loads = []
        for k in range(KD):
            sl = pl.ds(off + k * CH, CH)
            loads.append(pltpu.async_copy(src_hbm.at[sl], sidx[k], semi))
            loads.append(pltpu.async_copy(dst_hbm.at[sl], didx[k], semi))
        for d in loads:
            d.wait()
        adds = []
        for k in range(KD):
            adds.append(
                pltpu.async_copy(ohs_v, deg_sh.at[sidx[k]], sems, add=True))
            adds.append(
                pltpu.async_copy(ohd_v, deg_sh.at[didx[k]], sems, add=True))
        for d in adds:
            d.wait()
        return carry

    lax.fori_loop(0, NGD, body, 0)
    plsc.subcore_barrier()
    pltpu.sync_copy(deg_sh.at[rows], out_hbm.at[c, rows])


@functools.partial(
    pl.kernel,
    out_type=jax.ShapeDtypeStruct((NC, NP, D), jnp.float32),
    mesh=_mesh,
    scratch_types=(
        [pltpu.VMEM((CH,), jnp.int32)] * (2 * KS)
        + [pltpu.VMEM((CH,), jnp.int32)] * (2 * KS)
        + [pltpu.VMEM((CH, D), jnp.float32)] * (2 * KS)
        + [
            pltpu.VMEM_SHARED((NP, D), jnp.float32),
            pltpu.SemaphoreType.DMA,
            pltpu.SemaphoreType.DMA,
            pltpu.SemaphoreType.DMA,
            pltpu.SemaphoreType.DMA,
        ]
    ),
)
def _scatter_kernel(y_hbm, src_hbm, dst_hbm, zeros_hbm, out_hbm, *rest):
    sidx = rest[0:2 * KS]
    didx = rest[2 * KS:4 * KS]
    rows_v = rest[4 * KS:6 * KS]
    agg_sh, semi, semg, sems_a, sems_b = rest[6 * KS:]
    sems = (sems_a, sems_b)
    c = lax.axis_index("c")
    s = lax.axis_index("s")
    base = (s * NC + c) * EPT
    rows = pl.ds(s * RPT, RPT)
    pltpu.sync_copy(zeros_hbm, agg_sh.at[rows])
    plsc.subcore_barrier()

    def stage(off, half):
        base_k = half * KS
        loads = []
        for k in range(KS):
            sl = pl.ds(off + k * CH, CH)
            loads.append(
                pltpu.async_copy(src_hbm.at[sl], sidx[base_k + k], semi))
            loads.append(
                pltpu.async_copy(dst_hbm.at[sl], didx[base_k + k], semi))
        for d in loads:
            d.wait()
        gathers = [
            pltpu.async_copy(y_hbm.at[sidx[base_k + k]], rows_v[base_k + k],
                             semg)
            for k in range(KS)
        ]
        for d in gathers:
            d.wait()
        return [
            pltpu.async_copy(rows_v[base_k + k], agg_sh.at[didx[base_k + k]],
                             sems[half], add=True)
            for k in range(KS)
        ]

    def drain(half):
        for k in range(KS):
            pltpu.make_async_copy(rows_v[half * KS + k],
                                  agg_sh.at[didx[half * KS + k]],
                                  sems[half]).wait()

    stage(base, 0)

    def body(g, carry):
        off = base + g * (2 * KS * CH)
        stage(off + KS * CH, 1)
        drain(0)
        stage(off + 2 * KS * CH, 0)
        drain(1)
        return carry

    lax.fori_loop(0, NGS - 1, body, 0)
    offl = base + (NGS - 1) * (2 * KS * CH)
    stage(offl + KS * CH, 1)
    drain(0)
    drain(1)
    toff = base + NGS * (2 * KS * CH)
    pltpu.sync_copy(src_hbm.at[pl.ds(toff, CH)], sidx[0])
    pltpu.async_copy(y_hbm.at[sidx[0]], rows_v[0], semg).wait()
    pltpu.sync_copy(dst_hbm.at[pl.ds(toff, CH)], didx[0])
    pltpu.sync_copy(rows_v[0], agg_sh.at[didx[0]], add=True)
    plsc.subcore_barrier()
    pltpu.sync_copy(agg_sh.at[rows], out_hbm.at[c, rows])


def _tc0_body(x_ref, w_ref, hw_ref):
    hw_ref[...] = jnp.dot(x_ref[...], w_ref[...],
                          preferred_element_type=jnp.float32)


def _tc1_body(parts_ref, hw_ref, y_ref, nrm_ref):
    p = (parts_ref[0] + parts_ref[1])[:N]
    t_out = p[:, 0:16]
    t_in = p[:, 16:32]
    deg_out = jnp.maximum(jnp.sum(t_out, axis=1, keepdims=True), 1.0)
    deg_in = jnp.maximum(jnp.sum(t_in, axis=1, keepdims=True), 1.0)
    ns = lax.rsqrt(deg_out)
    nd = lax.rsqrt(deg_in)
    nrm_ref[:, 0:1] = ns
    nrm_ref[:, 1:2] = nd
    y_ref[...] = hw_ref[...] * ns


def _tc2_body(agg_ref, nrm_ref, b_ref, w_ref, y_ref):
    agg = (agg_ref[0] + agg_ref[1])[:N]
    ns = nrm_ref[:, 0:1]
    nd = nrm_ref[:, 1:2]
    h = jnp.maximum(agg * nd + b_ref[...][None, :], 0.0)
    hw = jnp.dot(h, w_ref[...], preferred_element_type=jnp.float32)
    y_ref[...] = hw * ns


def _tc3_body(agg_ref, nrm_ref, b_ref, out_ref):
    agg = (agg_ref[0] + agg_ref[1])[:N]
    nd = nrm_ref[:, 1:2]
    out_ref[...] = agg * nd + b_ref[...][None, :]


_tc0 = pl.pallas_call(
    _tc0_body,
    out_shape=jax.ShapeDtypeStruct((N, D), jnp.float32),
)

_tc1 = pl.pallas_call(
    _tc1_body,
    out_shape=(
        jax.ShapeDtypeStruct((N, D), jnp.float32),
        jax.ShapeDtypeStruct((N, 2), jnp.float32),
    ),
)

_tc2 = pl.pallas_call(
    _tc2_body,
    out_shape=jax.ShapeDtypeStruct((N, D), jnp.float32),
)

_tc3 = pl.pallas_call(
    _tc3_body,
    out_shape=jax.ShapeDtypeStruct((N, D), jnp.float32),
)


@jax.jit
def kernel(in_feat, edge_index, W1, b1, W2, b2):
    src = edge_index[0]
    dst = edge_index[1]
    eye = jnp.tile(jnp.eye(16, dtype=jnp.float32), (CH // 16, 1))
    oh_src = jnp.pad(eye, ((0, 0), (0, D - 16)))
    oh_dst = jnp.pad(eye, ((0, 0), (16, D - 32)))
    onehot = jnp.stack([oh_src, oh_dst])
    zerosd = jnp.zeros((RPT, D), jnp.float32)

    parts = _deg_kernel(src, dst, onehot, zerosd)
    hw1 = _tc0(in_feat, W1)
    y1, nrm = _tc1(parts, hw1)
    agg1 = _scatter_kernel(y1, src, dst, zerosd)
    y2 = _tc2(agg1, nrm, b1, W2)
    agg2 = _scatter_kernel(y2, src, dst, zerosd)
    return _tc3(agg2, nrm, b2)


# speedup vs baseline: 12.7667x; 1.0213x over previous
import functools

import jax
import jax.numpy as jnp
from jax import lax
from jax.experimental import pallas as pl
from jax.experimental.pallas import tpu as pltpu
from jax.experimental.pallas import tpu_sc as plsc

N = 10000
NP = 10240
E = 320000
D = 128

NC = 2
NS = 16
NW = NC * NS
EPT = E // NW
CH = 80
NCH = EPT // CH
KD = 5
NGD = NCH // KD
KS = 2
NGS = NCH // (2 * KS)
RPT = NP // NS

_mesh = plsc.VectorSubcoreMesh(core_axis_name="c", subcore_axis_name="s")


@functools.partial(
    pl.kernel,
    out_type=jax.ShapeDtypeStruct((NC, NP, D), jnp.float32),
    mesh=_mesh,
    scratch_types=(
        [pltpu.VMEM((CH, D), jnp.float32)] * 2
        + [pltpu.VMEM((CH,), jnp.int32)] * KD
        + [pltpu.VMEM((CH,), jnp.int32)] * KD
        + [
            pltpu.VMEM_SHARED((NP, D), jnp.float32),
            pltpu.SemaphoreType.DMA,
            pltpu.SemaphoreType.DMA,
        ]
    ),
)
def _deg_kernel(src_hbm, dst_hbm, onehot_hbm, zeros_hbm, out_hbm,
                ohs_v, ohd_v, *rest):
    sidx = rest[0:KD]
    didx = rest[KD:2 * KD]
    deg_sh, semi, sems = rest[2 * KD:]
    c = lax.axis_index("c")
    s = lax.axis_index("s")
    base = (s * NC + c) * EPT
    pltpu.sync_copy(onehot_hbm.at[0], ohs_v)
    pltpu.sync_copy(onehot_hbm.at[1], ohd_v)
    rows = pl.ds(s * RPT, RPT)
    pltpu.sync_copy(zeros_hbm, deg_sh.at[rows])
    plsc.subcore_barrier()

    def body(g, carry):
        off = base + g * (KD * CH)
        loads = []
        for k in range(KD):
            sl = pl.ds(off + k * CH, CH)
            loads.append(pltpu.async_copy(src_hbm.at[sl], sidx[k], semi))
            loads.append(pltpu.async_copy(dst_hbm.at[sl], didx[k], semi))
        for d in loads:
            d.wait()
        adds = []
        for k in range(KD):
            adds.append(
                pltpu.async_copy(ohs_v, deg_sh.at[sidx[k]], sems, add=True))
            adds.append(
                pltpu.async_copy(ohd_v, deg_sh.at[didx[k]], sems, add=True))
        for d in adds:
            d.wait()
        return carry

    lax.fori_loop(0, NGD, body, 0)
    plsc.subcore_barrier()
    pltpu.sync_copy(deg_sh.at[rows], out_hbm.at[c, rows])


@functools.partial(
    pl.kernel,
    out_type=jax.ShapeDtypeStruct((NC, NP, D), jnp.float32),
    mesh=_mesh,
    scratch_types=(
        [pltpu.VMEM((CH,), jnp.int32)] * (2 * KS)
        + [pltpu.VMEM((CH,), jnp.int32)] * (2 * KS)
        + [pltpu.VMEM((CH, D), jnp.float32)] * (2 * KS)
        + [
            pltpu.VMEM_SHARED((NP, D), jnp.float32),
            pltpu.SemaphoreType.DMA,
            pltpu.SemaphoreType.DMA,
            pltpu.SemaphoreType.DMA,
            pltpu.SemaphoreType.DMA,
        ]
    ),
)
def _scatter_kernel(y_hbm, src_hbm, dst_hbm, zeros_hbm, out_hbm, *rest):
    sidx = rest[0:2 * KS]
    didx = rest[2 * KS:4 * KS]
    rows_v = rest[4 * KS:6 * KS]
    agg_sh, semi, semg, sems_a, sems_b = rest[6 * KS:]
    sems = (sems_a, sems_b)
    c = lax.axis_index("c")
    s = lax.axis_index("s")
    base = (s * NC + c) * EPT
    rows = pl.ds(s * RPT, RPT)
    pltpu.sync_copy(zeros_hbm, agg_sh.at[rows])
    plsc.subcore_barrier()

    def stage(off, half):
        base_k = half * KS
        loads = []
        for k in range(KS):
            sl = pl.ds(off + k * CH, CH)
            loads.append(
                pltpu.async_copy(src_hbm.at[sl], sidx[base_k + k], semi))
            loads.append(
                pltpu.async_copy(dst_hbm.at[sl], didx[base_k + k], semi))
        for d in loads:
            d.wait()
        gathers = [
            pltpu.async_copy(y_hbm.at[sidx[base_k + k]], rows_v[base_k + k],
                             semg)
            for k in range(KS)
        ]
        for d in gathers:
            d.wait()
        return [
            pltpu.async_copy(rows_v[base_k + k], agg_sh.at[didx[base_k + k]],
                             sems[half], add=True)
            for k in range(KS)
        ]

    def drain(half):
        for k in range(KS):
            pltpu.make_async_copy(rows_v[half * KS + k],
                                  agg_sh.at[didx[half * KS + k]],
                                  sems[half]).wait()

    stage(base, 0)

    def body(g, carry):
        off = base + g * (2 * KS * CH)
        stage(off + KS * CH, 1)
        drain(0)
        stage(off + 2 * KS * CH, 0)
        drain(1)
        return carry

    lax.fori_loop(0, NGS - 1, body, 0)
    offl = base + (NGS - 1) * (2 * KS * CH)
    stage(offl + KS * CH, 1)
    drain(0)
    drain(1)
    toff = base + NGS * (2 * KS * CH)
    pltpu.sync_copy(src_hbm.at[pl.ds(toff, CH)], sidx[0])
    pltpu.async_copy(y_hbm.at[sidx[0]], rows_v[0], semg).wait()
    pltpu.sync_copy(dst_hbm.at[pl.ds(toff, CH)], didx[0])
    pltpu.sync_copy(rows_v[0], agg_sh.at[didx[0]], add=True)
    plsc.subcore_barrier()
    pltpu.sync_copy(agg_sh.at[rows], out_hbm.at[c, rows])


NB = 10
BR = N // NB


def _tc0_body(x_ref, w_ref, hw_ref):
    hw_ref[...] = jnp.dot(x_ref[...], w_ref[...],
                          preferred_element_type=jnp.float32)


def _tc1_body(parts_ref, hw_ref, y_ref, nrm_ref):
    p = parts_ref[0] + parts_ref[1]
    t_out = p[:, 0:16]
    t_in = p[:, 16:32]
    deg_out = jnp.maximum(jnp.sum(t_out, axis=1, keepdims=True), 1.0)
    deg_in = jnp.maximum(jnp.sum(t_in, axis=1, keepdims=True), 1.0)
    ns = lax.rsqrt(deg_out)
    nd = lax.rsqrt(deg_in)
    nrm_ref[:, 0:1] = ns
    nrm_ref[:, 1:2] = nd
    y_ref[...] = hw_ref[...] * ns


def _tc2_body(agg_ref, nrm_ref, b_ref, w_ref, y_ref):
    agg = agg_ref[0] + agg_ref[1]
    ns = nrm_ref[:, 0:1]
    nd = nrm_ref[:, 1:2]
    h = jnp.maximum(agg * nd + b_ref[...][None, :], 0.0)
    hw = jnp.dot(h, w_ref[...], preferred_element_type=jnp.float32)
    y_ref[...] = hw * ns


def _tc3_body(agg_ref, nrm_ref, b_ref, out_ref):
    agg = agg_ref[0] + agg_ref[1]
    nd = nrm_ref[:, 1:2]
    out_ref[...] = agg * nd + b_ref[...][None, :]


_row_spec = pl.BlockSpec((BR, D), lambda i: (i, 0))
_nrm_spec = pl.BlockSpec((BR, 2), lambda i: (i, 0))
_agg_spec = pl.BlockSpec((NC, BR, D), lambda i: (0, i, 0))
_w_spec = pl.BlockSpec((D, D), lambda i: (0, 0))
_b_spec = pl.BlockSpec((D,), lambda i: (0,))

_tc0 = pl.pallas_call(
    _tc0_body,
    grid=(NB,),
    in_specs=[_row_spec, _w_spec],
    out_specs=_row_spec,
    out_shape=jax.ShapeDtypeStruct((N, D), jnp.float32),
)

_tc1 = pl.pallas_call(
    _tc1_body,
    grid=(NB,),
    in_specs=[_agg_spec, _row_spec],
    out_specs=(_row_spec, _nrm_spec),
    out_shape=(
        jax.ShapeDtypeStruct((N, D), jnp.float32),
        jax.ShapeDtypeStruct((N, 2), jnp.float32),
    ),
)

_tc2 = pl.pallas_call(
    _tc2_body,
    grid=(NB,),
    in_specs=[_agg_spec, _nrm_spec, _b_spec, _w_spec],
    out_specs=_row_spec,
    out_shape=jax.ShapeDtypeStruct((N, D), jnp.float32),
)

_tc3 = pl.pallas_call(
    _tc3_body,
    grid=(NB,),
    in_specs=[_agg_spec, _nrm_spec, _b_spec],
    out_specs=_row_spec,
    out_shape=jax.ShapeDtypeStruct((N, D), jnp.float32),
)


@jax.jit
def kernel(in_feat, edge_index, W1, b1, W2, b2):
    src = edge_index[0]
    dst = edge_index[1]
    eye = jnp.tile(jnp.eye(16, dtype=jnp.float32), (CH // 16, 1))
    oh_src = jnp.pad(eye, ((0, 0), (0, D - 16)))
    oh_dst = jnp.pad(eye, ((0, 0), (16, D - 32)))
    onehot = jnp.stack([oh_src, oh_dst])
    zerosd = jnp.zeros((RPT, D), jnp.float32)

    parts = _deg_kernel(src, dst, onehot, zerosd)
    hw1 = _tc0(in_feat, W1)
    y1, nrm = _tc1(parts, hw1)
    agg1 = _scatter_kernel(y1, src, dst, zerosd)
    y2 = _tc2(agg1, nrm, b1, W2)
    agg2 = _scatter_kernel(y2, src, dst, zerosd)
    return _tc3(agg2, nrm, b2)
